# Initial kernel scaffold; baseline (speedup 1.0000x reference)
import jax
import jax.numpy as jnp
from jax.experimental import pallas as pl


def kernel(points, params):
    raise NotImplementedError("write your pallas kernel here")


# speedup vs baseline: 1.0019x; 1.0019x over previous
import functools
import jax
import jax.numpy as jnp
from jax.experimental import pallas as pl

_EPS = 1e-5
_NUM_CLASSES = 13


def _fold(plist):
    out = []
    for p in plist:
        s = p['gamma'] / jnp.sqrt(1.0 + _EPS)
        out.append((p['W'].T * s[None, :], p['beta']))
    return out


def _sqdist(src, dst):
    return (jnp.sum(src ** 2, -1)[..., :, None]
            + jnp.sum(dst ** 2, -1)[..., None, :]
            - 2.0 * jnp.matmul(src, jnp.swapaxes(dst, -1, -2)))


def _index_points(points, idx):
    b = points.shape[0]
    batch_idx = jnp.arange(b).reshape((b,) + (1,) * (idx.ndim - 1))
    return points[batch_idx, idx]


def _fps(xyz, npoint):
    b, n, _ = xyz.shape
    def body(state, _):
        distance, farthest = state
        centroid = jnp.take_along_axis(xyz, farthest[:, None, None], axis=1)
        dist = jnp.sum((xyz - centroid) ** 2, -1)
        distance = jnp.minimum(distance, dist)
        nxt = jnp.argmax(distance, -1).astype(jnp.int32)
        return (distance, nxt), farthest
    init = (jnp.full((b, n), 1e10, jnp.float32), jnp.zeros((b,), jnp.int32))
    _, cent = jax.lax.scan(body, init, None, length=npoint)
    return jnp.transpose(cent, (1, 0))


def _query_ball(radius, nsample, xyz, new_xyz):
    b, n, _ = xyz.shape
    s = new_xyz.shape[1]
    sqrdists = _sqdist(new_xyz, xyz)
    group_idx = jnp.broadcast_to(jnp.arange(n, dtype=jnp.int32), (b, s, n))
    group_idx = jnp.where(sqrdists > radius ** 2, n, group_idx)
    group_idx = jnp.sort(group_idx, axis=-1)[:, :, :nsample]
    group_first = group_idx[:, :, 0:1]
    group_idx = jnp.where(group_idx == n,
                          jnp.broadcast_to(group_first, group_idx.shape),
                          group_idx)
    return group_idx


def _mlp(x, folded):
    for wf, b in folded:
        x = jax.nn.relu(jnp.matmul(x, wf) + b)
    return x


def _set_abstraction(xyz, points, npoint, radius, nsample, folded):
    fps_idx = _fps(xyz, npoint)
    new_xyz = _index_points(xyz, fps_idx)
    idx = _query_ball(radius, nsample, xyz, new_xyz)
    grouped_xyz = _index_points(xyz, idx) - new_xyz[:, :, None, :]
    grouped = jnp.concatenate([grouped_xyz, _index_points(points, idx)], axis=-1)
    feat = _mlp(grouped, folded)
    return new_xyz, jnp.max(feat, axis=2)


def _feature_propagation(xyz1, xyz2, points1, points2, folded):
    dists = _sqdist(xyz1, xyz2)
    neg, idx = jax.lax.top_k(-dists, 3)
    d3 = jnp.maximum(-neg, 0.0)
    recip = 1.0 / (d3 + 1e-8)
    weight = recip / jnp.sum(recip, axis=2, keepdims=True)
    interpolated = jnp.sum(_index_points(points2, idx) * weight[..., None], axis=2)
    new_points = jnp.concatenate([points1, interpolated], axis=-1)
    return _mlp(new_points, folded)


def _head_body(x_ref, w1_ref, b1_ref, w2_ref, b2_ref, o_ref):
    x = x_ref[...]
    h = jax.nn.relu(jnp.dot(x, w1_ref[...], preferred_element_type=jnp.float32)
                    + b1_ref[...])
    o_ref[...] = (jnp.dot(h, w2_ref[...], preferred_element_type=jnp.float32)
                  + b2_ref[...])


def _head(x, w1, b1, w2, b2):
    bsz, n, c = x.shape
    xf = x.reshape(bsz * n, c)
    tile = 1024
    grid = (bsz * n // tile,)
    out = pl.pallas_call(
        _head_body,
        grid=grid,
        in_specs=[
            pl.BlockSpec((tile, c), lambda i: (i, 0)),
            pl.BlockSpec((c, c), lambda i: (0, 0)),
            pl.BlockSpec((1, c), lambda i: (0, 0)),
            pl.BlockSpec((c, _NUM_CLASSES), lambda i: (0, 0)),
            pl.BlockSpec((1, _NUM_CLASSES), lambda i: (0, 0)),
        ],
        out_specs=pl.BlockSpec((tile, _NUM_CLASSES), lambda i: (i, 0)),
        out_shape=jax.ShapeDtypeStruct((bsz * n, _NUM_CLASSES), jnp.float32),
    )(xf, w1, b1.reshape(1, -1), w2, b2.reshape(1, -1))
    return out.reshape(bsz, n, _NUM_CLASSES)


def kernel(points, params):
    pts = jnp.transpose(points, (0, 2, 1))
    l0_xyz = pts[:, :, :3]
    l0_points = pts[:, :, 3:]

    sa1 = _fold(params['sa1'])
    sa2 = _fold(params['sa2'])
    sa3 = _fold(params['sa3'])
    sa4 = _fold(params['sa4'])
    fp4 = _fold(params['fp4'])
    fp3 = _fold(params['fp3'])
    fp2 = _fold(params['fp2'])
    fp1 = _fold(params['fp1'])
    h1w, h1b = _fold([params['head1']])[0]

    l1_xyz, l1_points = _set_abstraction(l0_xyz, l0_points, 1024, 0.1, 32, sa1)
    l2_xyz, l2_points = _set_abstraction(l1_xyz, l1_points, 256, 0.2, 32, sa2)
    l3_xyz, l3_points = _set_abstraction(l2_xyz, l2_points, 64, 0.4, 32, sa3)
    l4_xyz, l4_points = _set_abstraction(l3_xyz, l3_points, 16, 0.8, 32, sa4)

    l3_points = _feature_propagation(l3_xyz, l4_xyz, l3_points, l4_points, fp4)
    l2_points = _feature_propagation(l2_xyz, l3_xyz, l2_points, l3_points, fp3)
    l1_points = _feature_propagation(l1_xyz, l2_xyz, l1_points, l2_points, fp2)
    l0_feat = _feature_propagation(l0_xyz, l1_xyz, l0_points, l1_points, fp1)

    return _head(l0_feat, h1w, h1b, params['conv2_W'].T, params['conv2_b'])


# speedup vs baseline: 1.1879x; 1.1857x over previous
import functools
import jax
import jax.numpy as jnp
from jax.experimental import pallas as pl

_EPS = 1e-5
_NUM_CLASSES = 13


def _fold(plist):
    out = []
    for p in plist:
        s = p['gamma'] / jnp.sqrt(1.0 + _EPS)
        out.append((p['W'].T * s[None, :], p['beta']))
    return out


def _sqdist(src, dst):
    return (jnp.sum(src ** 2, -1)[..., :, None]
            + jnp.sum(dst ** 2, -1)[..., None, :]
            - 2.0 * jnp.matmul(src, jnp.swapaxes(dst, -1, -2)))


def _index_points(points, idx):
    b = points.shape[0]
    batch_idx = jnp.arange(b).reshape((b,) + (1,) * (idx.ndim - 1))
    return points[batch_idx, idx]


def _fps_body(x0_ref, x1_ref, x2_ref, o_ref, *, npoint, n):
    X0 = x0_ref[0]
    X1 = x1_ref[0]
    X2 = x2_ref[0]
    rows, cols = X0.shape
    jr = jax.lax.broadcasted_iota(jnp.int32, (rows, cols), 0)
    jc = jax.lax.broadcasted_iota(jnp.int32, (rows, cols), 1)
    jidx = jr * cols + jc

    def body(k, carry):
        distance, far = carry
        oh = (jidx == far).astype(jnp.float32)
        c0 = jnp.sum(X0 * oh)
        c1 = jnp.sum(X1 * oh)
        c2 = jnp.sum(X2 * oh)
        cvec = jnp.concatenate(
            [jnp.reshape(c0, (1, 1)), jnp.reshape(c1, (1, 1)),
             jnp.reshape(c2, (1, 1))], axis=1)
        o_ref[0, pl.ds(k, 1), :] = cvec
        d0 = X0 - c0
        d1 = X1 - c1
        d2 = X2 - c2
        dist = (d0 * d0 + d1 * d1) + d2 * d2
        distance = jnp.minimum(distance, dist)
        m = jnp.max(distance)
        far2 = jnp.min(jnp.where(distance == m, jidx, n))
        return distance, far2

    dist0 = jnp.full((rows, cols), 1e10, jnp.float32)
    jax.lax.fori_loop(0, npoint, body, (dist0, jnp.int32(0)))


def _fps_new_xyz(xyz, npoint):
    b, n, _ = xyz.shape
    xr = jnp.transpose(xyz, (0, 2, 1)).reshape(b, 3, 8, n // 8)
    x0, x1, x2 = xr[:, 0], xr[:, 1], xr[:, 2]
    return pl.pallas_call(
        functools.partial(_fps_body, npoint=npoint, n=n),
        grid=(b,),
        in_specs=[pl.BlockSpec((1, 8, n // 8), lambda i: (i, 0, 0))] * 3,
        out_specs=pl.BlockSpec((1, npoint, 3), lambda i: (i, 0, 0)),
        out_shape=jax.ShapeDtypeStruct((b, npoint, 3), jnp.float32),
    )(x0, x1, x2)


def _query_ball(radius, nsample, xyz, new_xyz):
    b, n, _ = xyz.shape
    s = new_xyz.shape[1]
    sqrdists = _sqdist(new_xyz, xyz)
    group_idx = jnp.broadcast_to(jnp.arange(n, dtype=jnp.int32), (b, s, n))
    group_idx = jnp.where(sqrdists > radius ** 2, n, group_idx)
    group_idx = jnp.sort(group_idx, axis=-1)[:, :, :nsample]
    group_first = group_idx[:, :, 0:1]
    group_idx = jnp.where(group_idx == n,
                          jnp.broadcast_to(group_first, group_idx.shape),
                          group_idx)
    return group_idx


def _mlp(x, folded):
    for wf, b in folded:
        x = jax.nn.relu(jnp.matmul(x, wf) + b)
    return x


def _set_abstraction(xyz, points, npoint, radius, nsample, folded):
    new_xyz = _fps_new_xyz(xyz, npoint)
    idx = _query_ball(radius, nsample, xyz, new_xyz)
    grouped_xyz = _index_points(xyz, idx) - new_xyz[:, :, None, :]
    grouped = jnp.concatenate([grouped_xyz, _index_points(points, idx)], axis=-1)
    feat = _mlp(grouped, folded)
    return new_xyz, jnp.max(feat, axis=2)


def _feature_propagation(xyz1, xyz2, points1, points2, folded):
    dists = _sqdist(xyz1, xyz2)
    neg, idx = jax.lax.top_k(-dists, 3)
    d3 = jnp.maximum(-neg, 0.0)
    recip = 1.0 / (d3 + 1e-8)
    weight = recip / jnp.sum(recip, axis=2, keepdims=True)
    interpolated = jnp.sum(_index_points(points2, idx) * weight[..., None], axis=2)
    new_points = jnp.concatenate([points1, interpolated], axis=-1)
    return _mlp(new_points, folded)


def _head_body(x_ref, w1_ref, b1_ref, w2_ref, b2_ref, o_ref):
    x = x_ref[...]
    h = jax.nn.relu(jnp.dot(x, w1_ref[...], preferred_element_type=jnp.float32)
                    + b1_ref[...])
    o_ref[...] = (jnp.dot(h, w2_ref[...], preferred_element_type=jnp.float32)
                  + b2_ref[...])


def _head(x, w1, b1, w2, b2):
    bsz, n, c = x.shape
    xf = x.reshape(bsz * n, c)
    tile = 1024
    grid = (bsz * n // tile,)
    out = pl.pallas_call(
        _head_body,
        grid=grid,
        in_specs=[
            pl.BlockSpec((tile, c), lambda i: (i, 0)),
            pl.BlockSpec((c, c), lambda i: (0, 0)),
            pl.BlockSpec((1, c), lambda i: (0, 0)),
            pl.BlockSpec((c, _NUM_CLASSES), lambda i: (0, 0)),
            pl.BlockSpec((1, _NUM_CLASSES), lambda i: (0, 0)),
        ],
        out_specs=pl.BlockSpec((tile, _NUM_CLASSES), lambda i: (i, 0)),
        out_shape=jax.ShapeDtypeStruct((bsz * n, _NUM_CLASSES), jnp.float32),
    )(xf, w1, b1.reshape(1, -1), w2, b2.reshape(1, -1))
    return out.reshape(bsz, n, _NUM_CLASSES)


def kernel(points, params):
    pts = jnp.transpose(points, (0, 2, 1))
    l0_xyz = pts[:, :, :3]
    l0_points = pts[:, :, 3:]

    sa1 = _fold(params['sa1'])
    sa2 = _fold(params['sa2'])
    sa3 = _fold(params['sa3'])
    sa4 = _fold(params['sa4'])
    fp4 = _fold(params['fp4'])
    fp3 = _fold(params['fp3'])
    fp2 = _fold(params['fp2'])
    fp1 = _fold(params['fp1'])
    h1w, h1b = _fold([params['head1']])[0]

    l1_xyz, l1_points = _set_abstraction(l0_xyz, l0_points, 1024, 0.1, 32, sa1)
    l2_xyz, l2_points = _set_abstraction(l1_xyz, l1_points, 256, 0.2, 32, sa2)
    l3_xyz, l3_points = _set_abstraction(l2_xyz, l2_points, 64, 0.4, 32, sa3)
    l4_xyz, l4_points = _set_abstraction(l3_xyz, l3_points, 16, 0.8, 32, sa4)

    l3_points = _feature_propagation(l3_xyz, l4_xyz, l3_points, l4_points, fp4)
    l2_points = _feature_propagation(l2_xyz, l3_xyz, l2_points, l3_points, fp3)
    l1_points = _feature_propagation(l1_xyz, l2_xyz, l1_points, l2_points, fp2)
    l0_feat = _feature_propagation(l0_xyz, l1_xyz, l0_points, l1_points, fp1)

    return _head(l0_feat, h1w, h1b, params['conv2_W'].T, params['conv2_b'])


# speedup vs baseline: 1.9029x; 1.6019x over previous
import functools
import jax
import jax.numpy as jnp
from jax.experimental import pallas as pl
from jax.experimental.pallas import tpu as pltpu

_EPS = 1e-5
_NUM_CLASSES = 13


def _fold(plist):
    out = []
    for p in plist:
        s = p['gamma'] / jnp.sqrt(1.0 + _EPS)
        out.append((p['W'].T * s[None, :], p['beta']))
    return out


def _sqdist(src, dst):
    return (jnp.sum(src ** 2, -1)[..., :, None]
            + jnp.sum(dst ** 2, -1)[..., None, :]
            - 2.0 * jnp.matmul(src, jnp.swapaxes(dst, -1, -2)))


def _index_points(points, idx):
    b = points.shape[0]
    batch_idx = jnp.arange(b).reshape((b,) + (1,) * (idx.ndim - 1))
    return points[batch_idx, idx]


def _fps_body(x0_ref, x1_ref, x2_ref, o_ref, *, npoint, n):
    X0 = x0_ref[0]
    X1 = x1_ref[0]
    X2 = x2_ref[0]
    rows, cols = X0.shape
    jr = jax.lax.broadcasted_iota(jnp.int32, (rows, cols), 0)
    jc = jax.lax.broadcasted_iota(jnp.int32, (rows, cols), 1)
    jidx = jr * cols + jc

    def body(k, carry):
        distance, far = carry
        oh = (jidx == far).astype(jnp.float32)
        c0 = jnp.sum(X0 * oh)
        c1 = jnp.sum(X1 * oh)
        c2 = jnp.sum(X2 * oh)
        cvec = jnp.concatenate(
            [jnp.reshape(c0, (1, 1)), jnp.reshape(c1, (1, 1)),
             jnp.reshape(c2, (1, 1))], axis=1)
        o_ref[0, pl.ds(k, 1), :] = cvec
        d0 = X0 - c0
        d1 = X1 - c1
        d2 = X2 - c2
        dist = (d0 * d0 + d1 * d1) + d2 * d2
        distance = jnp.minimum(distance, dist)
        m = jnp.max(distance)
        far2 = jnp.min(jnp.where(distance == m, jidx, n))
        return distance, far2

    dist0 = jnp.full((rows, cols), 1e10, jnp.float32)
    jax.lax.fori_loop(0, npoint, body, (dist0, jnp.int32(0)))


def _fps_new_xyz(xyz, npoint):
    b, n, _ = xyz.shape
    xr = jnp.transpose(xyz, (0, 2, 1)).reshape(b, 3, 8, n // 8)
    x0, x1, x2 = xr[:, 0], xr[:, 1], xr[:, 2]
    return pl.pallas_call(
        functools.partial(_fps_body, npoint=npoint, n=n),
        grid=(b,),
        in_specs=[pl.BlockSpec((1, 8, n // 8), lambda i: (i, 0, 0))] * 3,
        out_specs=pl.BlockSpec((1, npoint, 3), lambda i: (i, 0, 0)),
        out_shape=jax.ShapeDtypeStruct((b, npoint, 3), jnp.float32),
    )(x0, x1, x2)


def _query_ball(radius, nsample, xyz, new_xyz):
    b, n, _ = xyz.shape
    s = new_xyz.shape[1]
    sqrdists = _sqdist(new_xyz, xyz)
    group_idx = jnp.broadcast_to(jnp.arange(n, dtype=jnp.int32), (b, s, n))
    group_idx = jnp.where(sqrdists > radius ** 2, n, group_idx)
    group_idx = jnp.sort(group_idx, axis=-1)[:, :, :nsample]
    group_first = group_idx[:, :, 0:1]
    group_idx = jnp.where(group_idx == n,
                          jnp.broadcast_to(group_first, group_idx.shape),
                          group_idx)
    return group_idx


def _mlp(x, folded):
    for wf, b in folded:
        x = jax.nn.relu(jnp.matmul(x, wf) + b)
    return x


def _linear_body(x_ref, w_ref, b_ref, o_ref, *, relu):
    y = jnp.dot(x_ref[...], w_ref[...], preferred_element_type=jnp.float32) + b_ref[...]
    o_ref[...] = jax.nn.relu(y) if relu else y


def _linear(x2d, w, b, relu=False, tile=512):
    rows, cin = x2d.shape
    cout = w.shape[1]
    return pl.pallas_call(
        functools.partial(_linear_body, relu=relu),
        grid=(rows // tile,),
        in_specs=[
            pl.BlockSpec((tile, cin), lambda i: (i, 0)),
            pl.BlockSpec((cin, cout), lambda i: (0, 0)),
            pl.BlockSpec((1, cout), lambda i: (0, 0)),
        ],
        out_specs=pl.BlockSpec((tile, cout), lambda i: (i, 0)),
        out_shape=jax.ShapeDtypeStruct((rows, cout), jnp.float32),
    )(x2d, w, b.reshape(1, -1))


def _sa_body(xyz_ref, nxyz_ref, a_ref, w1x_ref, w2_ref, b2_ref, w3_ref, b3_ref,
             o_ref, *, n, ns, s_t, r2):
    X = xyz_ref[0]
    C = nxyz_ref[0]
    x0, x1, x2 = X[0:1, :], X[1:2, :], X[2:3, :]
    c0, c1, c2 = C[:, 0:1], C[:, 1:2], C[:, 2:3]
    s_src = (c0 * c0 + c1 * c1) + c2 * c2
    s_dst = (x0 * x0 + x1 * x1) + x2 * x2
    cross = jnp.dot(C, X, preferred_element_type=jnp.float32)
    dist = s_src + s_dst - 2.0 * cross
    jidx = jax.lax.broadcasted_iota(jnp.int32, (1, n), 1)
    val = jnp.where(dist > r2, n, jnp.broadcast_to(jidx, (s_t, n)))

    cols = []
    for _ in range(ns):
        m = jnp.min(val, axis=1, keepdims=True)
        cols.append(m)
        val = jnp.where(val == m, n, val)
    first = cols[0]
    cols = [first] + [jnp.where(c == n, first, c) for c in cols[1:]]

    W1x = w1x_ref[...]
    d = c0 * W1x[0:1, :] + c1 * W1x[1:2, :] + c2 * W1x[2:3, :]

    jb = jnp.broadcast_to(jidx, (s_t, n))
    oh = jnp.concatenate([(jb == c).astype(jnp.float32) for c in cols], axis=0)
    G = jnp.dot(oh, a_ref[0], preferred_element_type=jnp.float32)
    D = jnp.concatenate([d] * ns, axis=0)
    act = jax.nn.relu(G - D)
    H = jax.nn.relu(jnp.dot(act, w2_ref[...], preferred_element_type=jnp.float32)
                    + b2_ref[...])
    F = jax.nn.relu(jnp.dot(H, w3_ref[...], preferred_element_type=jnp.float32)
                    + b3_ref[...])
    c3 = F.shape[-1]
    o_ref[0] = jnp.max(F.reshape(ns, s_t, c3), axis=0)


def _set_abstraction(xyz, points, npoint, radius, nsample, folded):
    b, n, _ = xyz.shape
    new_xyz = _fps_new_xyz(xyz, npoint)

    (w1, b1), (w2, b2), (w3, b3) = folded
    c1, c3 = w1.shape[1], w3.shape[1]
    feat_in = jnp.concatenate([xyz, points], axis=-1)
    cin = feat_in.shape[-1]
    a = _linear(feat_in.reshape(b * n, cin), w1, b1).reshape(b, n, c1)

    xr = jnp.transpose(xyz, (0, 2, 1))
    s_t = 8
    pooled = pl.pallas_call(
        functools.partial(_sa_body, n=n, ns=nsample, s_t=s_t,
                          r2=radius ** 2),
        grid=(b, npoint // s_t),
        in_specs=[
            pl.BlockSpec((1, 3, n), lambda i, j: (i, 0, 0)),
            pl.BlockSpec((1, s_t, 3), lambda i, j: (i, j, 0)),
            pl.BlockSpec((1, n, c1), lambda i, j: (i, 0, 0)),
            pl.BlockSpec((3, c1), lambda i, j: (0, 0)),
            pl.BlockSpec(w2.shape, lambda i, j: (0, 0)),
            pl.BlockSpec((1, w2.shape[1]), lambda i, j: (0, 0)),
            pl.BlockSpec(w3.shape, lambda i, j: (0, 0)),
            pl.BlockSpec((1, w3.shape[1]), lambda i, j: (0, 0)),
        ],
        out_specs=pl.BlockSpec((1, s_t, c3), lambda i, j: (i, j, 0)),
        out_shape=jax.ShapeDtypeStruct((b, npoint, c3), jnp.float32),
    )(xr, new_xyz, a, w1[:3], w2, b2.reshape(1, -1), w3, b3.reshape(1, -1))
    return new_xyz, pooled


def _feature_propagation(xyz1, xyz2, points1, points2, folded):
    dists = _sqdist(xyz1, xyz2)
    neg, idx = jax.lax.top_k(-dists, 3)
    d3 = jnp.maximum(-neg, 0.0)
    recip = 1.0 / (d3 + 1e-8)
    weight = recip / jnp.sum(recip, axis=2, keepdims=True)
    interpolated = jnp.sum(_index_points(points2, idx) * weight[..., None], axis=2)
    new_points = jnp.concatenate([points1, interpolated], axis=-1)
    return _mlp(new_points, folded)


def _head_body(x_ref, w1_ref, b1_ref, w2_ref, b2_ref, o_ref):
    x = x_ref[...]
    h = jax.nn.relu(jnp.dot(x, w1_ref[...], preferred_element_type=jnp.float32)
                    + b1_ref[...])
    o_ref[...] = (jnp.dot(h, w2_ref[...], preferred_element_type=jnp.float32)
                  + b2_ref[...])


def _head(x, w1, b1, w2, b2):
    bsz, n, c = x.shape
    xf = x.reshape(bsz * n, c)
    tile = 1024
    grid = (bsz * n // tile,)
    out = pl.pallas_call(
        _head_body,
        grid=grid,
        in_specs=[
            pl.BlockSpec((tile, c), lambda i: (i, 0)),
            pl.BlockSpec((c, c), lambda i: (0, 0)),
            pl.BlockSpec((1, c), lambda i: (0, 0)),
            pl.BlockSpec((c, _NUM_CLASSES), lambda i: (0, 0)),
            pl.BlockSpec((1, _NUM_CLASSES), lambda i: (0, 0)),
        ],
        out_specs=pl.BlockSpec((tile, _NUM_CLASSES), lambda i: (i, 0)),
        out_shape=jax.ShapeDtypeStruct((bsz * n, _NUM_CLASSES), jnp.float32),
    )(xf, w1, b1.reshape(1, -1), w2, b2.reshape(1, -1))
    return out.reshape(bsz, n, _NUM_CLASSES)


def kernel(points, params):
    pts = jnp.transpose(points, (0, 2, 1))
    l0_xyz = pts[:, :, :3]
    l0_points = pts[:, :, 3:]

    sa1 = _fold(params['sa1'])
    sa2 = _fold(params['sa2'])
    sa3 = _fold(params['sa3'])
    sa4 = _fold(params['sa4'])
    fp4 = _fold(params['fp4'])
    fp3 = _fold(params['fp3'])
    fp2 = _fold(params['fp2'])
    fp1 = _fold(params['fp1'])
    h1w, h1b = _fold([params['head1']])[0]

    l1_xyz, l1_points = _set_abstraction(l0_xyz, l0_points, 1024, 0.1, 32, sa1)
    l2_xyz, l2_points = _set_abstraction(l1_xyz, l1_points, 256, 0.2, 32, sa2)
    l3_xyz, l3_points = _set_abstraction(l2_xyz, l2_points, 64, 0.4, 32, sa3)
    l4_xyz, l4_points = _set_abstraction(l3_xyz, l3_points, 16, 0.8, 32, sa4)

    l3_points = _feature_propagation(l3_xyz, l4_xyz, l3_points, l4_points, fp4)
    l2_points = _feature_propagation(l2_xyz, l3_xyz, l2_points, l3_points, fp3)
    l1_points = _feature_propagation(l1_xyz, l2_xyz, l1_points, l2_points, fp2)
    l0_feat = _feature_propagation(l0_xyz, l1_xyz, l0_points, l1_points, fp1)

    return _head(l0_feat, h1w, h1b, params['conv2_W'].T, params['conv2_b'])


# speedup vs baseline: 2.8200x; 1.4819x over previous
import functools
import jax
import jax.numpy as jnp
from jax.experimental import pallas as pl
from jax.experimental.pallas import tpu as pltpu

_EPS = 1e-5
_NUM_CLASSES = 13


def _fold(plist):
    out = []
    for p in plist:
        s = p['gamma'] / jnp.sqrt(1.0 + _EPS)
        out.append((p['W'].T * s[None, :], p['beta']))
    return out


def _sqdist(src, dst):
    return (jnp.sum(src ** 2, -1)[..., :, None]
            + jnp.sum(dst ** 2, -1)[..., None, :]
            - 2.0 * jnp.matmul(src, jnp.swapaxes(dst, -1, -2)))


def _index_points(points, idx):
    b = points.shape[0]
    batch_idx = jnp.arange(b).reshape((b,) + (1,) * (idx.ndim - 1))
    return points[batch_idx, idx]


def _fps_body(x0_ref, x1_ref, x2_ref, o_ref, *, npoint, n):
    X0 = x0_ref[0]
    X1 = x1_ref[0]
    X2 = x2_ref[0]
    rows, cols = X0.shape
    jr = jax.lax.broadcasted_iota(jnp.int32, (rows, cols), 0)
    jc = jax.lax.broadcasted_iota(jnp.int32, (rows, cols), 1)
    jidx = jr * cols + jc

    def body(k, carry):
        distance, far = carry
        oh = (jidx == far).astype(jnp.float32)
        c0 = jnp.sum(X0 * oh)
        c1 = jnp.sum(X1 * oh)
        c2 = jnp.sum(X2 * oh)
        cvec = jnp.concatenate(
            [jnp.reshape(c0, (1, 1)), jnp.reshape(c1, (1, 1)),
             jnp.reshape(c2, (1, 1))], axis=1)
        o_ref[0, pl.ds(k, 1), :] = cvec
        d0 = X0 - c0
        d1 = X1 - c1
        d2 = X2 - c2
        dist = (d0 * d0 + d1 * d1) + d2 * d2
        distance = jnp.minimum(distance, dist)
        m = jnp.max(distance)
        far2 = jnp.min(jnp.where(distance == m, jidx, n))
        return distance, far2

    dist0 = jnp.full((rows, cols), 1e10, jnp.float32)
    jax.lax.fori_loop(0, npoint, body, (dist0, jnp.int32(0)))


def _fps_new_xyz(xyz, npoint):
    b, n, _ = xyz.shape
    xr = jnp.transpose(xyz, (0, 2, 1)).reshape(b, 3, 8, n // 8)
    x0, x1, x2 = xr[:, 0], xr[:, 1], xr[:, 2]
    return pl.pallas_call(
        functools.partial(_fps_body, npoint=npoint, n=n),
        grid=(b,),
        in_specs=[pl.BlockSpec((1, 8, n // 8), lambda i: (i, 0, 0))] * 3,
        out_specs=pl.BlockSpec((1, npoint, 3), lambda i: (i, 0, 0)),
        out_shape=jax.ShapeDtypeStruct((b, npoint, 3), jnp.float32),
    )(x0, x1, x2)


def _query_ball(radius, nsample, xyz, new_xyz):
    b, n, _ = xyz.shape
    s = new_xyz.shape[1]
    sqrdists = _sqdist(new_xyz, xyz)
    group_idx = jnp.broadcast_to(jnp.arange(n, dtype=jnp.int32), (b, s, n))
    group_idx = jnp.where(sqrdists > radius ** 2, n, group_idx)
    group_idx = jnp.sort(group_idx, axis=-1)[:, :, :nsample]
    group_first = group_idx[:, :, 0:1]
    group_idx = jnp.where(group_idx == n,
                          jnp.broadcast_to(group_first, group_idx.shape),
                          group_idx)
    return group_idx


def _mlp(x, folded):
    for wf, b in folded:
        x = jax.nn.relu(jnp.matmul(x, wf) + b)
    return x


def _linear_body(x_ref, w_ref, b_ref, o_ref, *, relu):
    y = jnp.dot(x_ref[...], w_ref[...], preferred_element_type=jnp.float32) + b_ref[...]
    o_ref[...] = jax.nn.relu(y) if relu else y


def _linear(x2d, w, b, relu=False, tile=512):
    rows, cin = x2d.shape
    cout = w.shape[1]
    return pl.pallas_call(
        functools.partial(_linear_body, relu=relu),
        grid=(rows // tile,),
        in_specs=[
            pl.BlockSpec((tile, cin), lambda i: (i, 0)),
            pl.BlockSpec((cin, cout), lambda i: (0, 0)),
            pl.BlockSpec((1, cout), lambda i: (0, 0)),
        ],
        out_specs=pl.BlockSpec((tile, cout), lambda i: (i, 0)),
        out_shape=jax.ShapeDtypeStruct((rows, cout), jnp.float32),
    )(x2d, w, b.reshape(1, -1))


def _sa_body(xyz_ref, nxyz_ref, a_ref, w1x_ref, w2_ref, b2_ref, w3_ref, b3_ref,
             o_ref, *, n, ns, s_t, r2):
    X = xyz_ref[0]
    C = nxyz_ref[0]
    x0, x1, x2 = X[0:1, :], X[1:2, :], X[2:3, :]
    c0, c1, c2 = C[:, 0:1], C[:, 1:2], C[:, 2:3]
    s_src = (c0 * c0 + c1 * c1) + c2 * c2
    s_dst = (x0 * x0 + x1 * x1) + x2 * x2
    cross = jnp.dot(C, X, preferred_element_type=jnp.float32)
    dist = s_src + s_dst - 2.0 * cross
    jidx = jax.lax.broadcasted_iota(jnp.int32, (1, n), 1)
    val = jnp.where(dist > r2, n, jnp.broadcast_to(jidx, (s_t, n)))

    cols = []
    for _ in range(ns):
        m = jnp.min(val, axis=1, keepdims=True)
        cols.append(m)
        val = jnp.where(val == m, n, val)
    first = cols[0]
    cols = [first] + [jnp.where(c == n, first, c) for c in cols[1:]]

    W1x = w1x_ref[...]
    d = c0 * W1x[0:1, :] + c1 * W1x[1:2, :] + c2 * W1x[2:3, :]

    jb = jnp.broadcast_to(jidx, (s_t, n))
    oh = jnp.concatenate([(jb == c).astype(jnp.float32) for c in cols], axis=0)
    G = jnp.dot(oh, a_ref[0], preferred_element_type=jnp.float32)
    D = jnp.concatenate([d] * ns, axis=0)
    act = jax.nn.relu(G - D)
    H = jax.nn.relu(jnp.dot(act, w2_ref[...], preferred_element_type=jnp.float32)
                    + b2_ref[...])
    F = jax.nn.relu(jnp.dot(H, w3_ref[...], preferred_element_type=jnp.float32)
                    + b3_ref[...])
    c3 = F.shape[-1]
    o_ref[0] = jnp.max(F.reshape(ns, s_t, c3), axis=0)


def _set_abstraction(xyz, points, npoint, radius, nsample, folded):
    b, n, _ = xyz.shape
    new_xyz = _fps_new_xyz(xyz, npoint)

    (w1, b1), (w2, b2), (w3, b3) = folded
    c1, c3 = w1.shape[1], w3.shape[1]
    feat_in = jnp.concatenate([xyz, points], axis=-1)
    cin = feat_in.shape[-1]
    a = _linear(feat_in.reshape(b * n, cin), w1, b1).reshape(b, n, c1)

    xr = jnp.transpose(xyz, (0, 2, 1))
    s_t = 8
    pooled = pl.pallas_call(
        functools.partial(_sa_body, n=n, ns=nsample, s_t=s_t,
                          r2=radius ** 2),
        grid=(b, npoint // s_t),
        in_specs=[
            pl.BlockSpec((1, 3, n), lambda i, j: (i, 0, 0)),
            pl.BlockSpec((1, s_t, 3), lambda i, j: (i, j, 0)),
            pl.BlockSpec((1, n, c1), lambda i, j: (i, 0, 0)),
            pl.BlockSpec((3, c1), lambda i, j: (0, 0)),
            pl.BlockSpec(w2.shape, lambda i, j: (0, 0)),
            pl.BlockSpec((1, w2.shape[1]), lambda i, j: (0, 0)),
            pl.BlockSpec(w3.shape, lambda i, j: (0, 0)),
            pl.BlockSpec((1, w3.shape[1]), lambda i, j: (0, 0)),
        ],
        out_specs=pl.BlockSpec((1, s_t, c3), lambda i, j: (i, j, 0)),
        out_shape=jax.ShapeDtypeStruct((b, npoint, c3), jnp.float32),
    )(xr, new_xyz, a, w1[:3], w2, b2.reshape(1, -1), w3, b3.reshape(1, -1))
    return new_xyz, pooled


def _feature_propagation(xyz1, xyz2, points1, points2, folded):
    dists = _sqdist(xyz1, xyz2)
    neg, idx = jax.lax.top_k(-dists, 3)
    d3 = jnp.maximum(-neg, 0.0)
    recip = 1.0 / (d3 + 1e-8)
    weight = recip / jnp.sum(recip, axis=2, keepdims=True)
    interpolated = jnp.sum(_index_points(points2, idx) * weight[..., None], axis=2)
    new_points = jnp.concatenate([points1, interpolated], axis=-1)
    return _mlp(new_points, folded)


def _fp_body(*refs, n2, s_t, last_relu):
    xyz1_ref, xyz2_ref, p1_ref, p2_ref = refs[0:4]
    wrefs = refs[4:-1]
    o_ref = refs[-1]
    C = xyz1_ref[0]
    X = xyz2_ref[0]
    x0, x1, x2 = X[0:1, :], X[1:2, :], X[2:3, :]
    c0, c1, c2 = C[:, 0:1], C[:, 1:2], C[:, 2:3]
    s_src = (c0 * c0 + c1 * c1) + c2 * c2
    s_dst = (x0 * x0 + x1 * x1) + x2 * x2
    dist = s_src + s_dst - 2.0 * jnp.dot(C, X, preferred_element_type=jnp.float32)
    jidx = jax.lax.broadcasted_iota(jnp.int32, (1, n2), 1)
    jb = jnp.broadcast_to(jidx, (s_t, n2))

    ms, ims = [], []
    d = dist
    for _ in range(3):
        m = jnp.min(d, axis=1, keepdims=True)
        im = jnp.min(jnp.where(d == m, jb, n2), axis=1, keepdims=True)
        ms.append(m)
        ims.append(im)
        d = jnp.where(jb == im, jnp.float32(1e30), d)

    recips = [1.0 / (jnp.maximum(m, 0.0) + 1e-8) for m in ms]
    rsum = (recips[0] + recips[1]) + recips[2]
    u = ((jb == ims[0]).astype(jnp.float32) * (recips[0] / rsum)
         + (jb == ims[1]).astype(jnp.float32) * (recips[1] / rsum)
         + (jb == ims[2]).astype(jnp.float32) * (recips[2] / rsum))
    interp = jnp.dot(u, p2_ref[0], preferred_element_type=jnp.float32)

    x = jnp.concatenate([p1_ref[0], interp], axis=1)
    nl = len(wrefs) // 2
    for i in range(nl):
        y = (jnp.dot(x, wrefs[2 * i][...], preferred_element_type=jnp.float32)
             + wrefs[2 * i + 1][...])
        x = y if (i == nl - 1 and not last_relu) else jax.nn.relu(y)
    o_ref[0] = x


def _fp_pallas(xyz1, xyz2, points1, points2, layers, s_t, last_relu=True):
    b, n1, _ = xyz1.shape
    n2 = xyz2.shape[1]
    cp1 = points1.shape[-1]
    c2 = points2.shape[-1]
    x2r = jnp.transpose(xyz2, (0, 2, 1))
    flat = []
    wspecs = []
    for w, bb in layers:
        flat += [w, bb.reshape(1, -1)]
        wspecs += [pl.BlockSpec(w.shape, lambda i, j: (0, 0)),
                   pl.BlockSpec((1, w.shape[1]), lambda i, j: (0, 0))]
    cout = layers[-1][0].shape[1]
    return pl.pallas_call(
        functools.partial(_fp_body, n2=n2, s_t=s_t, last_relu=last_relu),
        grid=(b, n1 // s_t),
        in_specs=[
            pl.BlockSpec((1, s_t, 3), lambda i, j: (i, j, 0)),
            pl.BlockSpec((1, 3, n2), lambda i, j: (i, 0, 0)),
            pl.BlockSpec((1, s_t, cp1), lambda i, j: (i, j, 0)),
            pl.BlockSpec((1, n2, c2), lambda i, j: (i, 0, 0)),
        ] + wspecs,
        out_specs=pl.BlockSpec((1, s_t, cout), lambda i, j: (i, j, 0)),
        out_shape=jax.ShapeDtypeStruct((b, n1, cout), jnp.float32),
    )(xyz1, x2r, points1, points2, *flat)


def _head_body(x_ref, w1_ref, b1_ref, w2_ref, b2_ref, o_ref):
    x = x_ref[...]
    h = jax.nn.relu(jnp.dot(x, w1_ref[...], preferred_element_type=jnp.float32)
                    + b1_ref[...])
    o_ref[...] = (jnp.dot(h, w2_ref[...], preferred_element_type=jnp.float32)
                  + b2_ref[...])


def _head(x, w1, b1, w2, b2):
    bsz, n, c = x.shape
    xf = x.reshape(bsz * n, c)
    tile = 1024
    grid = (bsz * n // tile,)
    out = pl.pallas_call(
        _head_body,
        grid=grid,
        in_specs=[
            pl.BlockSpec((tile, c), lambda i: (i, 0)),
            pl.BlockSpec((c, c), lambda i: (0, 0)),
            pl.BlockSpec((1, c), lambda i: (0, 0)),
            pl.BlockSpec((c, _NUM_CLASSES), lambda i: (0, 0)),
            pl.BlockSpec((1, _NUM_CLASSES), lambda i: (0, 0)),
        ],
        out_specs=pl.BlockSpec((tile, _NUM_CLASSES), lambda i: (i, 0)),
        out_shape=jax.ShapeDtypeStruct((bsz * n, _NUM_CLASSES), jnp.float32),
    )(xf, w1, b1.reshape(1, -1), w2, b2.reshape(1, -1))
    return out.reshape(bsz, n, _NUM_CLASSES)


def kernel(points, params):
    pts = jnp.transpose(points, (0, 2, 1))
    l0_xyz = pts[:, :, :3]
    l0_points = pts[:, :, 3:]

    sa1 = _fold(params['sa1'])
    sa2 = _fold(params['sa2'])
    sa3 = _fold(params['sa3'])
    sa4 = _fold(params['sa4'])
    fp4 = _fold(params['fp4'])
    fp3 = _fold(params['fp3'])
    fp2 = _fold(params['fp2'])
    fp1 = _fold(params['fp1'])
    h1w, h1b = _fold([params['head1']])[0]

    l1_xyz, l1_points = _set_abstraction(l0_xyz, l0_points, 1024, 0.1, 32, sa1)
    l2_xyz, l2_points = _set_abstraction(l1_xyz, l1_points, 256, 0.2, 32, sa2)
    l3_xyz, l3_points = _set_abstraction(l2_xyz, l2_points, 64, 0.4, 32, sa3)
    l4_xyz, l4_points = _set_abstraction(l3_xyz, l3_points, 16, 0.8, 32, sa4)

    l3_points = _fp_pallas(l3_xyz, l4_xyz, l3_points, l4_points, fp4, s_t=64)
    l2_points = _fp_pallas(l2_xyz, l3_xyz, l2_points, l3_points, fp3, s_t=128)
    l1_points = _fp_pallas(l1_xyz, l2_xyz, l1_points, l2_points, fp2, s_t=256)
    tail = fp1 + [(h1w, h1b), (params['conv2_W'].T, params['conv2_b'])]
    return _fp_pallas(l0_xyz, l1_xyz, l0_points, l1_points, tail, s_t=256,
                      last_relu=False)


# speedup vs baseline: 4.8162x; 1.7079x over previous
import functools
import jax
import jax.numpy as jnp
from jax.experimental import pallas as pl
from jax.experimental.pallas import tpu as pltpu

_EPS = 1e-5
_NUM_CLASSES = 13


def _fold(plist):
    out = []
    for p in plist:
        s = p['gamma'] / jnp.sqrt(1.0 + _EPS)
        out.append((p['W'].T * s[None, :], p['beta']))
    return out


def _sqdist(src, dst):
    return (jnp.sum(src ** 2, -1)[..., :, None]
            + jnp.sum(dst ** 2, -1)[..., None, :]
            - 2.0 * jnp.matmul(src, jnp.swapaxes(dst, -1, -2)))


def _index_points(points, idx):
    b = points.shape[0]
    batch_idx = jnp.arange(b).reshape((b,) + (1,) * (idx.ndim - 1))
    return points[batch_idx, idx]


def _fps_body(x0_ref, x1_ref, x2_ref, o_ref, *, npoint, n):
    X0 = x0_ref[0]
    X1 = x1_ref[0]
    X2 = x2_ref[0]
    rows, cols = X0.shape
    jr = jax.lax.broadcasted_iota(jnp.int32, (rows, cols), 0)
    jc = jax.lax.broadcasted_iota(jnp.int32, (rows, cols), 1)
    jidx = jr * cols + jc

    def body(k, carry):
        distance, far = carry
        oh = (jidx == far).astype(jnp.float32)
        c0 = jnp.sum(X0 * oh)
        c1 = jnp.sum(X1 * oh)
        c2 = jnp.sum(X2 * oh)
        cvec = jnp.concatenate(
            [jnp.reshape(c0, (1, 1)), jnp.reshape(c1, (1, 1)),
             jnp.reshape(c2, (1, 1))], axis=1)
        o_ref[0, pl.ds(k, 1), :] = cvec
        d0 = X0 - c0
        d1 = X1 - c1
        d2 = X2 - c2
        dist = (d0 * d0 + d1 * d1) + d2 * d2
        distance = jnp.minimum(distance, dist)
        m = jnp.max(distance)
        far2 = jnp.min(jnp.where(distance == m, jidx, n))
        return distance, far2

    dist0 = jnp.full((rows, cols), 1e10, jnp.float32)
    jax.lax.fori_loop(0, npoint, body, (dist0, jnp.int32(0)))


def _fps_new_xyz(xyz, npoint):
    b, n, _ = xyz.shape
    xr = jnp.transpose(xyz, (0, 2, 1)).reshape(b, 3, 8, n // 8)
    x0, x1, x2 = xr[:, 0], xr[:, 1], xr[:, 2]
    return pl.pallas_call(
        functools.partial(_fps_body, npoint=npoint, n=n),
        grid=(b,),
        in_specs=[pl.BlockSpec((1, 8, n // 8), lambda i: (i, 0, 0))] * 3,
        out_specs=pl.BlockSpec((1, npoint, 3), lambda i: (i, 0, 0)),
        out_shape=jax.ShapeDtypeStruct((b, npoint, 3), jnp.float32),
    )(x0, x1, x2)


def _query_ball(radius, nsample, xyz, new_xyz):
    b, n, _ = xyz.shape
    s = new_xyz.shape[1]
    sqrdists = _sqdist(new_xyz, xyz)
    group_idx = jnp.broadcast_to(jnp.arange(n, dtype=jnp.int32), (b, s, n))
    group_idx = jnp.where(sqrdists > radius ** 2, n, group_idx)
    group_idx = jnp.sort(group_idx, axis=-1)[:, :, :nsample]
    group_first = group_idx[:, :, 0:1]
    group_idx = jnp.where(group_idx == n,
                          jnp.broadcast_to(group_first, group_idx.shape),
                          group_idx)
    return group_idx


def _mlp(x, folded):
    for wf, b in folded:
        x = jax.nn.relu(jnp.matmul(x, wf) + b)
    return x


def _linear_body(x_ref, w_ref, b_ref, o_ref, *, relu):
    y = jnp.dot(x_ref[...], w_ref[...], preferred_element_type=jnp.float32) + b_ref[...]
    o_ref[...] = jax.nn.relu(y) if relu else y


def _linear(x2d, w, b, relu=False, tile=512):
    rows, cin = x2d.shape
    cout = w.shape[1]
    return pl.pallas_call(
        functools.partial(_linear_body, relu=relu),
        grid=(rows // tile,),
        in_specs=[
            pl.BlockSpec((tile, cin), lambda i: (i, 0)),
            pl.BlockSpec((cin, cout), lambda i: (0, 0)),
            pl.BlockSpec((1, cout), lambda i: (0, 0)),
        ],
        out_specs=pl.BlockSpec((tile, cout), lambda i: (i, 0)),
        out_shape=jax.ShapeDtypeStruct((rows, cout), jnp.float32),
    )(x2d, w, b.reshape(1, -1))


def _sa_body(xyz_ref, nxyz_ref, a_ref, w1x_ref, w2_ref, b2_ref, w3_ref, b3_ref,
             o_ref, *, n, ns, s_t, r2):
    X = xyz_ref[0]
    C = nxyz_ref[0]
    x0, x1, x2 = X[0:1, :], X[1:2, :], X[2:3, :]
    c0, c1, c2 = C[:, 0:1], C[:, 1:2], C[:, 2:3]
    s_src = (c0 * c0 + c1 * c1) + c2 * c2
    s_dst = (x0 * x0 + x1 * x1) + x2 * x2
    cross = jnp.dot(C, X, preferred_element_type=jnp.float32)
    dist = s_src + s_dst - 2.0 * cross
    jidx = jax.lax.broadcasted_iota(jnp.int32, (1, n), 1)
    val = jnp.where(dist > r2, n, jnp.broadcast_to(jidx, (s_t, n)))

    cols = []
    for _ in range(ns):
        m = jnp.min(val, axis=1, keepdims=True)
        cols.append(m)
        val = jnp.where(val == m, n, val)
    first = cols[0]
    cols = [first] + [jnp.where(c == n, first, c) for c in cols[1:]]

    W1x = w1x_ref[...]
    d = c0 * W1x[0:1, :] + c1 * W1x[1:2, :] + c2 * W1x[2:3, :]

    jb = jnp.broadcast_to(jidx, (s_t, n))
    oh = jnp.concatenate([(jb == c).astype(jnp.float32) for c in cols], axis=0)
    G = jnp.dot(oh, a_ref[0], preferred_element_type=jnp.float32)
    D = jnp.concatenate([d] * ns, axis=0)
    act = jax.nn.relu(G - D)
    H = jax.nn.relu(jnp.dot(act, w2_ref[...], preferred_element_type=jnp.float32)
                    + b2_ref[...])
    F = jax.nn.relu(jnp.dot(H, w3_ref[...], preferred_element_type=jnp.float32)
                    + b3_ref[...])
    c3 = F.shape[-1]
    o_ref[0] = jnp.max(F.reshape(ns, s_t, c3), axis=0)


def _set_abstraction(xyz, points, npoint, radius, nsample, folded):
    b, n, _ = xyz.shape
    new_xyz = _fps_new_xyz(xyz, npoint)

    (w1, b1), (w2, b2), (w3, b3) = folded
    c1, c3 = w1.shape[1], w3.shape[1]
    feat_in = jnp.concatenate([xyz, points], axis=-1)
    cin = feat_in.shape[-1]
    a = _linear(feat_in.reshape(b * n, cin), w1, b1).reshape(b, n, c1)

    xr = jnp.transpose(xyz, (0, 2, 1))
    s_t = min(32, npoint)
    pooled = pl.pallas_call(
        functools.partial(_sa_body, n=n, ns=nsample, s_t=s_t,
                          r2=radius ** 2),
        grid=(b, npoint // s_t),
        in_specs=[
            pl.BlockSpec((1, 3, n), lambda i, j: (i, 0, 0)),
            pl.BlockSpec((1, s_t, 3), lambda i, j: (i, j, 0)),
            pl.BlockSpec((1, n, c1), lambda i, j: (i, 0, 0)),
            pl.BlockSpec((3, c1), lambda i, j: (0, 0)),
            pl.BlockSpec(w2.shape, lambda i, j: (0, 0)),
            pl.BlockSpec((1, w2.shape[1]), lambda i, j: (0, 0)),
            pl.BlockSpec(w3.shape, lambda i, j: (0, 0)),
            pl.BlockSpec((1, w3.shape[1]), lambda i, j: (0, 0)),
        ],
        out_specs=pl.BlockSpec((1, s_t, c3), lambda i, j: (i, j, 0)),
        out_shape=jax.ShapeDtypeStruct((b, npoint, c3), jnp.float32),
    )(xr, new_xyz, a, w1[:3], w2, b2.reshape(1, -1), w3, b3.reshape(1, -1))
    return new_xyz, pooled


def _feature_propagation(xyz1, xyz2, points1, points2, folded):
    dists = _sqdist(xyz1, xyz2)
    neg, idx = jax.lax.top_k(-dists, 3)
    d3 = jnp.maximum(-neg, 0.0)
    recip = 1.0 / (d3 + 1e-8)
    weight = recip / jnp.sum(recip, axis=2, keepdims=True)
    interpolated = jnp.sum(_index_points(points2, idx) * weight[..., None], axis=2)
    new_points = jnp.concatenate([points1, interpolated], axis=-1)
    return _mlp(new_points, folded)


def _fp_body(*refs, n2, s_t, last_relu):
    xyz1_ref, xyz2_ref, p1_ref, p2_ref = refs[0:4]
    wrefs = refs[4:-1]
    o_ref = refs[-1]
    C = xyz1_ref[0]
    X = xyz2_ref[0]
    x0, x1, x2 = X[0:1, :], X[1:2, :], X[2:3, :]
    c0, c1, c2 = C[:, 0:1], C[:, 1:2], C[:, 2:3]
    s_src = (c0 * c0 + c1 * c1) + c2 * c2
    s_dst = (x0 * x0 + x1 * x1) + x2 * x2
    dist = s_src + s_dst - 2.0 * jnp.dot(C, X, preferred_element_type=jnp.float32)
    jidx = jax.lax.broadcasted_iota(jnp.int32, (1, n2), 1)
    jb = jnp.broadcast_to(jidx, (s_t, n2))

    ms, ims = [], []
    d = dist
    for _ in range(3):
        m = jnp.min(d, axis=1, keepdims=True)
        im = jnp.min(jnp.where(d == m, jb, n2), axis=1, keepdims=True)
        ms.append(m)
        ims.append(im)
        d = jnp.where(jb == im, jnp.float32(1e30), d)

    recips = [1.0 / (jnp.maximum(m, 0.0) + 1e-8) for m in ms]
    rsum = (recips[0] + recips[1]) + recips[2]
    u = ((jb == ims[0]).astype(jnp.float32) * (recips[0] / rsum)
         + (jb == ims[1]).astype(jnp.float32) * (recips[1] / rsum)
         + (jb == ims[2]).astype(jnp.float32) * (recips[2] / rsum))
    interp = jnp.dot(u, p2_ref[0], preferred_element_type=jnp.float32)

    x = jnp.concatenate([p1_ref[0], interp], axis=1)
    nl = len(wrefs) // 2
    for i in range(nl):
        y = (jnp.dot(x, wrefs[2 * i][...], preferred_element_type=jnp.float32)
             + wrefs[2 * i + 1][...])
        x = y if (i == nl - 1 and not last_relu) else jax.nn.relu(y)
    o_ref[0] = x


def _fp_pallas(xyz1, xyz2, points1, points2, layers, s_t, last_relu=True):
    b, n1, _ = xyz1.shape
    n2 = xyz2.shape[1]
    cp1 = points1.shape[-1]
    c2 = points2.shape[-1]
    x2r = jnp.transpose(xyz2, (0, 2, 1))
    flat = []
    wspecs = []
    for w, bb in layers:
        flat += [w, bb.reshape(1, -1)]
        wspecs += [pl.BlockSpec(w.shape, lambda i, j: (0, 0)),
                   pl.BlockSpec((1, w.shape[1]), lambda i, j: (0, 0))]
    cout = layers[-1][0].shape[1]
    return pl.pallas_call(
        functools.partial(_fp_body, n2=n2, s_t=s_t, last_relu=last_relu),
        grid=(b, n1 // s_t),
        in_specs=[
            pl.BlockSpec((1, s_t, 3), lambda i, j: (i, j, 0)),
            pl.BlockSpec((1, 3, n2), lambda i, j: (i, 0, 0)),
            pl.BlockSpec((1, s_t, cp1), lambda i, j: (i, j, 0)),
            pl.BlockSpec((1, n2, c2), lambda i, j: (i, 0, 0)),
        ] + wspecs,
        out_specs=pl.BlockSpec((1, s_t, cout), lambda i, j: (i, j, 0)),
        out_shape=jax.ShapeDtypeStruct((b, n1, cout), jnp.float32),
    )(xyz1, x2r, points1, points2, *flat)


def _head_body(x_ref, w1_ref, b1_ref, w2_ref, b2_ref, o_ref):
    x = x_ref[...]
    h = jax.nn.relu(jnp.dot(x, w1_ref[...], preferred_element_type=jnp.float32)
                    + b1_ref[...])
    o_ref[...] = (jnp.dot(h, w2_ref[...], preferred_element_type=jnp.float32)
                  + b2_ref[...])


def _head(x, w1, b1, w2, b2):
    bsz, n, c = x.shape
    xf = x.reshape(bsz * n, c)
    tile = 1024
    grid = (bsz * n // tile,)
    out = pl.pallas_call(
        _head_body,
        grid=grid,
        in_specs=[
            pl.BlockSpec((tile, c), lambda i: (i, 0)),
            pl.BlockSpec((c, c), lambda i: (0, 0)),
            pl.BlockSpec((1, c), lambda i: (0, 0)),
            pl.BlockSpec((c, _NUM_CLASSES), lambda i: (0, 0)),
            pl.BlockSpec((1, _NUM_CLASSES), lambda i: (0, 0)),
        ],
        out_specs=pl.BlockSpec((tile, _NUM_CLASSES), lambda i: (i, 0)),
        out_shape=jax.ShapeDtypeStruct((bsz * n, _NUM_CLASSES), jnp.float32),
    )(xf, w1, b1.reshape(1, -1), w2, b2.reshape(1, -1))
    return out.reshape(bsz, n, _NUM_CLASSES)


def kernel(points, params):
    pts = jnp.transpose(points, (0, 2, 1))
    l0_xyz = pts[:, :, :3]
    l0_points = pts[:, :, 3:]

    sa1 = _fold(params['sa1'])
    sa2 = _fold(params['sa2'])
    sa3 = _fold(params['sa3'])
    sa4 = _fold(params['sa4'])
    fp4 = _fold(params['fp4'])
    fp3 = _fold(params['fp3'])
    fp2 = _fold(params['fp2'])
    fp1 = _fold(params['fp1'])
    h1w, h1b = _fold([params['head1']])[0]

    l1_xyz, l1_points = _set_abstraction(l0_xyz, l0_points, 1024, 0.1, 32, sa1)
    l2_xyz, l2_points = _set_abstraction(l1_xyz, l1_points, 256, 0.2, 32, sa2)
    l3_xyz, l3_points = _set_abstraction(l2_xyz, l2_points, 64, 0.4, 32, sa3)
    l4_xyz, l4_points = _set_abstraction(l3_xyz, l3_points, 16, 0.8, 32, sa4)

    l3_points = _fp_pallas(l3_xyz, l4_xyz, l3_points, l4_points, fp4, s_t=64)
    l2_points = _fp_pallas(l2_xyz, l3_xyz, l2_points, l3_points, fp3, s_t=128)
    l1_points = _fp_pallas(l1_xyz, l2_xyz, l1_points, l2_points, fp2, s_t=256)
    tail = fp1 + [(h1w, h1b), (params['conv2_W'].T, params['conv2_b'])]
    return _fp_pallas(l0_xyz, l1_xyz, l0_points, l1_points, tail, s_t=256,
                      last_relu=False)


# speedup vs baseline: 5.4067x; 1.1226x over previous
import functools
import jax
import jax.numpy as jnp
from jax.experimental import pallas as pl
from jax.experimental.pallas import tpu as pltpu

_EPS = 1e-5
_NUM_CLASSES = 13


def _fold(plist):
    out = []
    for p in plist:
        s = p['gamma'] / jnp.sqrt(1.0 + _EPS)
        out.append((p['W'].T * s[None, :], p['beta']))
    return out


def _sqdist(src, dst):
    return (jnp.sum(src ** 2, -1)[..., :, None]
            + jnp.sum(dst ** 2, -1)[..., None, :]
            - 2.0 * jnp.matmul(src, jnp.swapaxes(dst, -1, -2)))


def _index_points(points, idx):
    b = points.shape[0]
    batch_idx = jnp.arange(b).reshape((b,) + (1,) * (idx.ndim - 1))
    return points[batch_idx, idx]


def _fps_body(x0_ref, x1_ref, x2_ref, o_ref, *, npoint, n):
    X0 = x0_ref[0]
    X1 = x1_ref[0]
    X2 = x2_ref[0]
    rows, cols = X0.shape
    jr = jax.lax.broadcasted_iota(jnp.int32, (rows, cols), 0)
    jc = jax.lax.broadcasted_iota(jnp.int32, (rows, cols), 1)
    jidx = jr * cols + jc

    def body(k, carry):
        distance, far = carry
        oh = (jidx == far).astype(jnp.float32)
        c0 = jnp.sum(X0 * oh)
        c1 = jnp.sum(X1 * oh)
        c2 = jnp.sum(X2 * oh)
        cvec = jnp.concatenate(
            [jnp.reshape(c0, (1, 1)), jnp.reshape(c1, (1, 1)),
             jnp.reshape(c2, (1, 1))], axis=1)
        o_ref[0, pl.ds(k, 1), :] = cvec
        d0 = X0 - c0
        d1 = X1 - c1
        d2 = X2 - c2
        dist = (d0 * d0 + d1 * d1) + d2 * d2
        distance = jnp.minimum(distance, dist)
        m = jnp.max(distance)
        far2 = jnp.min(jnp.where(distance == m, jidx, n))
        return distance, far2

    dist0 = jnp.full((rows, cols), 1e10, jnp.float32)
    jax.lax.fori_loop(0, npoint, body, (dist0, jnp.int32(0)))


def _fps_new_xyz(xyz, npoint):
    b, n, _ = xyz.shape
    xr = jnp.transpose(xyz, (0, 2, 1)).reshape(b, 3, 8, n // 8)
    x0, x1, x2 = xr[:, 0], xr[:, 1], xr[:, 2]
    return pl.pallas_call(
        functools.partial(_fps_body, npoint=npoint, n=n),
        grid=(b,),
        in_specs=[pl.BlockSpec((1, 8, n // 8), lambda i: (i, 0, 0))] * 3,
        out_specs=pl.BlockSpec((1, npoint, 3), lambda i: (i, 0, 0)),
        out_shape=jax.ShapeDtypeStruct((b, npoint, 3), jnp.float32),
    )(x0, x1, x2)


def _query_ball(radius, nsample, xyz, new_xyz):
    b, n, _ = xyz.shape
    s = new_xyz.shape[1]
    sqrdists = _sqdist(new_xyz, xyz)
    group_idx = jnp.broadcast_to(jnp.arange(n, dtype=jnp.int32), (b, s, n))
    group_idx = jnp.where(sqrdists > radius ** 2, n, group_idx)
    group_idx = jnp.sort(group_idx, axis=-1)[:, :, :nsample]
    group_first = group_idx[:, :, 0:1]
    group_idx = jnp.where(group_idx == n,
                          jnp.broadcast_to(group_first, group_idx.shape),
                          group_idx)
    return group_idx


def _mlp(x, folded):
    for wf, b in folded:
        x = jax.nn.relu(jnp.matmul(x, wf) + b)
    return x


def _linear_body(x_ref, w_ref, b_ref, o_ref, *, relu):
    y = jnp.dot(x_ref[...], w_ref[...], preferred_element_type=jnp.float32) + b_ref[...]
    o_ref[...] = jax.nn.relu(y) if relu else y


def _linear(x2d, w, b, relu=False, tile=512):
    rows, cin = x2d.shape
    cout = w.shape[1]
    return pl.pallas_call(
        functools.partial(_linear_body, relu=relu),
        grid=(rows // tile,),
        in_specs=[
            pl.BlockSpec((tile, cin), lambda i: (i, 0)),
            pl.BlockSpec((cin, cout), lambda i: (0, 0)),
            pl.BlockSpec((1, cout), lambda i: (0, 0)),
        ],
        out_specs=pl.BlockSpec((tile, cout), lambda i: (i, 0)),
        out_shape=jax.ShapeDtypeStruct((rows, cout), jnp.float32),
    )(x2d, w, b.reshape(1, -1))


def _sa_body(xyz_ref, nxyz_ref, a_ref, w1x_ref, w2_ref, b2_ref, w3_ref, b3_ref,
             o_ref, *, n, ns, s_t, r2):
    X = xyz_ref[0]
    C = nxyz_ref[0]
    x0, x1, x2 = X[0:1, :], X[1:2, :], X[2:3, :]
    c0, c1, c2 = C[:, 0:1], C[:, 1:2], C[:, 2:3]
    s_src = (c0 * c0 + c1 * c1) + c2 * c2
    s_dst = (x0 * x0 + x1 * x1) + x2 * x2
    cross = jnp.dot(C, X, preferred_element_type=jnp.float32)
    dist = s_src + s_dst - 2.0 * cross
    jidx = jax.lax.broadcasted_iota(jnp.int32, (1, n), 1)
    val = jnp.where(dist > r2, n, jnp.broadcast_to(jidx, (s_t, n)))

    cols = []
    for _ in range(ns):
        m = jnp.min(val, axis=1, keepdims=True)
        cols.append(m)
        val = jnp.where(val == m, n, val)
    first = cols[0]
    cols = [first] + [jnp.where(c == n, first, c) for c in cols[1:]]

    W1x = w1x_ref[...]
    d = c0 * W1x[0:1, :] + c1 * W1x[1:2, :] + c2 * W1x[2:3, :]

    jb = jnp.broadcast_to(jidx, (s_t, n))
    oh = jnp.concatenate([(jb == c).astype(jnp.float32) for c in cols], axis=0)
    G = jnp.dot(oh, a_ref[0], preferred_element_type=jnp.float32)
    D = jnp.concatenate([d] * ns, axis=0)
    act = jax.nn.relu(G - D)
    H = jax.nn.relu(jnp.dot(act, w2_ref[...], preferred_element_type=jnp.float32)
                    + b2_ref[...])
    F = jax.nn.relu(jnp.dot(H, w3_ref[...], preferred_element_type=jnp.float32)
                    + b3_ref[...])
    c3 = F.shape[-1]
    o_ref[0] = jnp.max(F.reshape(ns, s_t, c3), axis=0)


def _set_abstraction(xyz, points, npoint, radius, nsample, folded):
    b, n, _ = xyz.shape
    new_xyz = _fps_new_xyz(xyz, npoint)

    (w1, b1), (w2, b2), (w3, b3) = folded
    c1, c3 = w1.shape[1], w3.shape[1]
    feat_in = jnp.concatenate([xyz, points], axis=-1)
    cin = feat_in.shape[-1]
    a = _linear(feat_in.reshape(b * n, cin), w1, b1).reshape(b, n, c1)

    xr = jnp.transpose(xyz, (0, 2, 1))
    s_t = min(64, npoint)
    pooled = pl.pallas_call(
        functools.partial(_sa_body, n=n, ns=nsample, s_t=s_t,
                          r2=radius ** 2),
        grid=(b, npoint // s_t),
        in_specs=[
            pl.BlockSpec((1, 3, n), lambda i, j: (i, 0, 0)),
            pl.BlockSpec((1, s_t, 3), lambda i, j: (i, j, 0)),
            pl.BlockSpec((1, n, c1), lambda i, j: (i, 0, 0)),
            pl.BlockSpec((3, c1), lambda i, j: (0, 0)),
            pl.BlockSpec(w2.shape, lambda i, j: (0, 0)),
            pl.BlockSpec((1, w2.shape[1]), lambda i, j: (0, 0)),
            pl.BlockSpec(w3.shape, lambda i, j: (0, 0)),
            pl.BlockSpec((1, w3.shape[1]), lambda i, j: (0, 0)),
        ],
        out_specs=pl.BlockSpec((1, s_t, c3), lambda i, j: (i, j, 0)),
        out_shape=jax.ShapeDtypeStruct((b, npoint, c3), jnp.float32),
    )(xr, new_xyz, a, w1[:3], w2, b2.reshape(1, -1), w3, b3.reshape(1, -1))
    return new_xyz, pooled


def _feature_propagation(xyz1, xyz2, points1, points2, folded):
    dists = _sqdist(xyz1, xyz2)
    neg, idx = jax.lax.top_k(-dists, 3)
    d3 = jnp.maximum(-neg, 0.0)
    recip = 1.0 / (d3 + 1e-8)
    weight = recip / jnp.sum(recip, axis=2, keepdims=True)
    interpolated = jnp.sum(_index_points(points2, idx) * weight[..., None], axis=2)
    new_points = jnp.concatenate([points1, interpolated], axis=-1)
    return _mlp(new_points, folded)


def _fp_body(*refs, n2, s_t, last_relu):
    xyz1_ref, xyz2_ref, p1_ref, p2_ref = refs[0:4]
    wrefs = refs[4:-1]
    o_ref = refs[-1]
    C = xyz1_ref[0]
    X = xyz2_ref[0]
    x0, x1, x2 = X[0:1, :], X[1:2, :], X[2:3, :]
    c0, c1, c2 = C[:, 0:1], C[:, 1:2], C[:, 2:3]
    s_src = (c0 * c0 + c1 * c1) + c2 * c2
    s_dst = (x0 * x0 + x1 * x1) + x2 * x2
    dist = s_src + s_dst - 2.0 * jnp.dot(C, X, preferred_element_type=jnp.float32)
    jidx = jax.lax.broadcasted_iota(jnp.int32, (1, n2), 1)
    jb = jnp.broadcast_to(jidx, (s_t, n2))

    ms, ims = [], []
    d = dist
    for _ in range(3):
        m = jnp.min(d, axis=1, keepdims=True)
        im = jnp.min(jnp.where(d == m, jb, n2), axis=1, keepdims=True)
        ms.append(m)
        ims.append(im)
        d = jnp.where(jb == im, jnp.float32(1e30), d)

    recips = [1.0 / (jnp.maximum(m, 0.0) + 1e-8) for m in ms]
    rsum = (recips[0] + recips[1]) + recips[2]
    u = ((jb == ims[0]).astype(jnp.float32) * (recips[0] / rsum)
         + (jb == ims[1]).astype(jnp.float32) * (recips[1] / rsum)
         + (jb == ims[2]).astype(jnp.float32) * (recips[2] / rsum))
    interp = jnp.dot(u, p2_ref[0], preferred_element_type=jnp.float32)

    x = jnp.concatenate([p1_ref[0], interp], axis=1)
    nl = len(wrefs) // 2
    for i in range(nl):
        y = (jnp.dot(x, wrefs[2 * i][...], preferred_element_type=jnp.float32)
             + wrefs[2 * i + 1][...])
        x = y if (i == nl - 1 and not last_relu) else jax.nn.relu(y)
    o_ref[0] = x


def _fp_pallas(xyz1, xyz2, points1, points2, layers, s_t, last_relu=True):
    b, n1, _ = xyz1.shape
    n2 = xyz2.shape[1]
    cp1 = points1.shape[-1]
    c2 = points2.shape[-1]
    x2r = jnp.transpose(xyz2, (0, 2, 1))
    flat = []
    wspecs = []
    for w, bb in layers:
        flat += [w, bb.reshape(1, -1)]
        wspecs += [pl.BlockSpec(w.shape, lambda i, j: (0, 0)),
                   pl.BlockSpec((1, w.shape[1]), lambda i, j: (0, 0))]
    cout = layers[-1][0].shape[1]
    return pl.pallas_call(
        functools.partial(_fp_body, n2=n2, s_t=s_t, last_relu=last_relu),
        grid=(b, n1 // s_t),
        in_specs=[
            pl.BlockSpec((1, s_t, 3), lambda i, j: (i, j, 0)),
            pl.BlockSpec((1, 3, n2), lambda i, j: (i, 0, 0)),
            pl.BlockSpec((1, s_t, cp1), lambda i, j: (i, j, 0)),
            pl.BlockSpec((1, n2, c2), lambda i, j: (i, 0, 0)),
        ] + wspecs,
        out_specs=pl.BlockSpec((1, s_t, cout), lambda i, j: (i, j, 0)),
        out_shape=jax.ShapeDtypeStruct((b, n1, cout), jnp.float32),
    )(xyz1, x2r, points1, points2, *flat)


def _head_body(x_ref, w1_ref, b1_ref, w2_ref, b2_ref, o_ref):
    x = x_ref[...]
    h = jax.nn.relu(jnp.dot(x, w1_ref[...], preferred_element_type=jnp.float32)
                    + b1_ref[...])
    o_ref[...] = (jnp.dot(h, w2_ref[...], preferred_element_type=jnp.float32)
                  + b2_ref[...])


def _head(x, w1, b1, w2, b2):
    bsz, n, c = x.shape
    xf = x.reshape(bsz * n, c)
    tile = 1024
    grid = (bsz * n // tile,)
    out = pl.pallas_call(
        _head_body,
        grid=grid,
        in_specs=[
            pl.BlockSpec((tile, c), lambda i: (i, 0)),
            pl.BlockSpec((c, c), lambda i: (0, 0)),
            pl.BlockSpec((1, c), lambda i: (0, 0)),
            pl.BlockSpec((c, _NUM_CLASSES), lambda i: (0, 0)),
            pl.BlockSpec((1, _NUM_CLASSES), lambda i: (0, 0)),
        ],
        out_specs=pl.BlockSpec((tile, _NUM_CLASSES), lambda i: (i, 0)),
        out_shape=jax.ShapeDtypeStruct((bsz * n, _NUM_CLASSES), jnp.float32),
    )(xf, w1, b1.reshape(1, -1), w2, b2.reshape(1, -1))
    return out.reshape(bsz, n, _NUM_CLASSES)


def kernel(points, params):
    pts = jnp.transpose(points, (0, 2, 1))
    l0_xyz = pts[:, :, :3]
    l0_points = pts[:, :, 3:]

    sa1 = _fold(params['sa1'])
    sa2 = _fold(params['sa2'])
    sa3 = _fold(params['sa3'])
    sa4 = _fold(params['sa4'])
    fp4 = _fold(params['fp4'])
    fp3 = _fold(params['fp3'])
    fp2 = _fold(params['fp2'])
    fp1 = _fold(params['fp1'])
    h1w, h1b = _fold([params['head1']])[0]

    l1_xyz, l1_points = _set_abstraction(l0_xyz, l0_points, 1024, 0.1, 32, sa1)
    l2_xyz, l2_points = _set_abstraction(l1_xyz, l1_points, 256, 0.2, 32, sa2)
    l3_xyz, l3_points = _set_abstraction(l2_xyz, l2_points, 64, 0.4, 32, sa3)
    l4_xyz, l4_points = _set_abstraction(l3_xyz, l3_points, 16, 0.8, 32, sa4)

    l3_points = _fp_pallas(l3_xyz, l4_xyz, l3_points, l4_points, fp4, s_t=64)
    l2_points = _fp_pallas(l2_xyz, l3_xyz, l2_points, l3_points, fp3, s_t=128)
    l1_points = _fp_pallas(l1_xyz, l2_xyz, l1_points, l2_points, fp2, s_t=256)
    tail = fp1 + [(h1w, h1b), (params['conv2_W'].T, params['conv2_b'])]
    return _fp_pallas(l0_xyz, l1_xyz, l0_points, l1_points, tail, s_t=256,
                      last_relu=False)
